# baseline (speedup 1.0000x reference)
import functools

import jax
import jax.numpy as jnp
from jax import lax
from jax.experimental import pallas as pl
from jax.experimental.pallas import tpu as pltpu, tpu_sc as plsc

N_ROWS = 16384
N_COLS = 3328
N_SLICES = 26
SLICE_LEN = 128
VECS = SLICE_LEN // 16
CHUNK_ROWS = 4
NUM_WORKERS = 32
CHUNKS_PER_WORKER = N_ROWS // (NUM_WORKERS * CHUNK_ROWS)
IDX_PAD = 48


def _pick(vec, idx):
    return lax.gather(
        vec,
        idx[:, None],
        lax.GatherDimensionNumbers(
            offset_dims=(), collapsed_slice_dims=(0,), start_index_map=(0,)
        ),
        (1,),
        mode=lax.GatherScatterMode.PROMISE_IN_BOUNDS,
    )


def _slice_body(inp_hbm, idx_hbm, out_hbm, idx_v, win_v, out_v, gsem, ssem):
    wid = lax.axis_index("s") * 2 + lax.axis_index("c")
    pltpu.sync_copy(idx_hbm, idx_v)
    row_base = wid * CHUNKS_PER_WORKER * CHUNK_ROWS
    lane = lax.iota(jnp.int32, 16)

    def r0_of(t):
        return row_base + t * CHUNK_ROWS

    def start_gather(b, t):
        pltpu.async_copy(
            inp_hbm.at[pl.ds(r0_of(t), CHUNK_ROWS), :], win_v.at[b], gsem.at[b]
        )

    for bb in range(4):
        start_gather(bb, bb)

    def chunk(k, carry):
        for b in range(4):
            t = 4 * k + b
            pltpu.make_async_copy(
                inp_hbm.at[pl.ds(r0_of(t), CHUNK_ROWS), :],
                win_v.at[b],
                gsem.at[b],
            ).wait()

            @pl.when(t >= 4)
            def _():
                pltpu.make_async_copy(
                    out_v.at[b],
                    out_hbm.at[:, pl.ds(0, CHUNK_ROWS), :],
                    ssem.at[b],
                ).wait()

            @plsc.parallel_loop(0, N_SLICES)
            def one_slice(i):
                s = idx_v[pl.ds(i, 16)][0]
                base = pl.multiple_of(s & ~15, 16)
                rem = s & 15
                ia_lo = (lane + rem) & 15
                from_a = (lane + rem) < 16
                for r in range(CHUNK_ROWS):
                    a = win_v[b, r, pl.ds(base, 16)]
                    ga = _pick(a, ia_lo)
                    for j in range(VECS):
                        bvec = win_v[
                            b, r, pl.ds(pl.multiple_of(base + 16 * j + 16, 16), 16)
                        ]
                        gb = _pick(bvec, ia_lo)
                        out_v[b, i, r, pl.ds(16 * j, 16)] = jnp.where(
                            from_a, ga, gb
                        )
                        ga = gb

            pltpu.async_copy(
                out_v.at[b],
                out_hbm.at[:, pl.ds(r0_of(t), CHUNK_ROWS), :],
                ssem.at[b],
            )

            @pl.when(t + 4 < CHUNKS_PER_WORKER)
            def _():
                start_gather(b, t + 4)

        return carry

    lax.fori_loop(0, CHUNKS_PER_WORKER // 4, chunk, 0)

    for b in range(4):
        pltpu.make_async_copy(
            out_v.at[b],
            out_hbm.at[:, pl.ds(0, CHUNK_ROWS), :],
            ssem.at[b],
        ).wait()


def kernel(input_tensor, slices_index, slice_len):
    idx_padded = jnp.zeros((IDX_PAD,), jnp.int32).at[:N_SLICES].set(slices_index)
    mesh = plsc.VectorSubcoreMesh(core_axis_name="c", subcore_axis_name="s")
    run = pl.kernel(
        _slice_body,
        out_type=jax.ShapeDtypeStruct((N_SLICES, N_ROWS, SLICE_LEN), jnp.float32),
        mesh=mesh,
        scratch_types=[
            pltpu.VMEM((IDX_PAD,), jnp.int32),
            pltpu.VMEM((4, CHUNK_ROWS, N_COLS), jnp.float32),
            pltpu.VMEM((4, N_SLICES, CHUNK_ROWS, SLICE_LEN), jnp.float32),
            pltpu.SemaphoreType.DMA((4,)),
            pltpu.SemaphoreType.DMA((4,)),
        ],
    )
    return run(input_tensor, idx_padded)


# speedup vs baseline: 1.0949x; 1.0949x over previous
import functools

import jax
import jax.numpy as jnp
from jax import lax
from jax.experimental import pallas as pl
from jax.experimental.pallas import tpu as pltpu, tpu_sc as plsc

N_ROWS = 16384
N_COLS = 3328
N_SLICES = 26
SLICE_LEN = 128
VECS = SLICE_LEN // 16
CHUNK_ROWS = 8
NUM_WORKERS = 32
CHUNKS_PER_WORKER = N_ROWS // (NUM_WORKERS * CHUNK_ROWS)
IDX_PAD = 48


def _pick(vec, idx):
    return lax.gather(
        vec,
        idx[:, None],
        lax.GatherDimensionNumbers(
            offset_dims=(), collapsed_slice_dims=(0,), start_index_map=(0,)
        ),
        (1,),
        mode=lax.GatherScatterMode.PROMISE_IN_BOUNDS,
    )


def _slice_body(inp_hbm, idx_hbm, out_hbm, idx_v, win_v, out_v, gsem, ssem):
    wid = lax.axis_index("s") * 2 + lax.axis_index("c")
    pltpu.sync_copy(idx_hbm, idx_v)
    row_base = wid * CHUNKS_PER_WORKER * CHUNK_ROWS
    lane = lax.iota(jnp.int32, 16)

    def r0_of(t):
        return row_base + t * CHUNK_ROWS

    def start_gather(b, t):
        pltpu.async_copy(
            inp_hbm.at[pl.ds(r0_of(t), CHUNK_ROWS), :], win_v.at[b], gsem.at[b]
        )

    start_gather(0, 0)
    start_gather(1, 1)

    def chunk(k, carry):
        for b in range(2):
            t = 2 * k + b
            pltpu.make_async_copy(
                inp_hbm.at[pl.ds(r0_of(t), CHUNK_ROWS), :],
                win_v.at[b],
                gsem.at[b],
            ).wait()

            @pl.when(t >= 2)
            def _():
                pltpu.make_async_copy(
                    out_v.at[b],
                    out_hbm.at[:, pl.ds(0, CHUNK_ROWS), :],
                    ssem.at[b],
                ).wait()

            @plsc.parallel_loop(0, N_SLICES, unroll=2)
            def one_slice(i):
                s = idx_v[pl.ds(i, 16)][0]
                base = pl.multiple_of(s & ~15, 16)
                rem = s & 15
                ia_lo = (lane + rem) & 15
                from_a = (lane + rem) < 16
                for r in range(CHUNK_ROWS):
                    a = win_v[b, r, pl.ds(base, 16)]
                    ga = _pick(a, ia_lo)
                    for j in range(VECS):
                        bvec = win_v[
                            b, r, pl.ds(pl.multiple_of(base + 16 * j + 16, 16), 16)
                        ]
                        gb = _pick(bvec, ia_lo)
                        out_v[b, i, r, pl.ds(16 * j, 16)] = jnp.where(
                            from_a, ga, gb
                        )
                        ga = gb

            pltpu.async_copy(
                out_v.at[b],
                out_hbm.at[:, pl.ds(r0_of(t), CHUNK_ROWS), :],
                ssem.at[b],
            )

            @pl.when(t + 2 < CHUNKS_PER_WORKER)
            def _():
                start_gather(b, t + 2)

        return carry

    lax.fori_loop(0, CHUNKS_PER_WORKER // 2, chunk, 0)

    for b in range(2):
        pltpu.make_async_copy(
            out_v.at[b],
            out_hbm.at[:, pl.ds(0, CHUNK_ROWS), :],
            ssem.at[b],
        ).wait()


def kernel(input_tensor, slices_index, slice_len):
    idx_padded = jnp.zeros((IDX_PAD,), jnp.int32).at[:N_SLICES].set(slices_index)
    mesh = plsc.VectorSubcoreMesh(core_axis_name="c", subcore_axis_name="s")
    run = pl.kernel(
        _slice_body,
        out_type=jax.ShapeDtypeStruct((N_SLICES, N_ROWS, SLICE_LEN), jnp.float32),
        mesh=mesh,
        scratch_types=[
            pltpu.VMEM((IDX_PAD,), jnp.int32),
            pltpu.VMEM((2, CHUNK_ROWS, N_COLS), jnp.float32),
            pltpu.VMEM((2, N_SLICES, CHUNK_ROWS, SLICE_LEN), jnp.float32),
            pltpu.SemaphoreType.DMA((2,)),
            pltpu.SemaphoreType.DMA((2,)),
        ],
    )
    return run(input_tensor, idx_padded)


# speedup vs baseline: 1.1065x; 1.0106x over previous
import functools

import jax
import jax.numpy as jnp
from jax import lax
from jax.experimental import pallas as pl
from jax.experimental.pallas import tpu as pltpu, tpu_sc as plsc

N_ROWS = 16384
N_COLS = 3328
N_SLICES = 26
SLICE_LEN = 128
VECS = SLICE_LEN // 16
CHUNK_ROWS = 8
NUM_WORKERS = 32
CHUNKS_PER_WORKER = N_ROWS // (NUM_WORKERS * CHUNK_ROWS)
IDX_PAD = 48


def _pick(vec, idx):
    return lax.gather(
        vec,
        idx[:, None],
        lax.GatherDimensionNumbers(
            offset_dims=(), collapsed_slice_dims=(0,), start_index_map=(0,)
        ),
        (1,),
        mode=lax.GatherScatterMode.PROMISE_IN_BOUNDS,
    )


def _slice_body(inp_hbm, idx_hbm, out_hbm, idx_v, win_v, out_v, gsem, ssem):
    wid = lax.axis_index("s") * 2 + lax.axis_index("c")
    pltpu.sync_copy(idx_hbm, idx_v)
    row_base = wid * CHUNKS_PER_WORKER * CHUNK_ROWS
    lane = lax.iota(jnp.int32, 16)

    def r0_of(t):
        return row_base + t * CHUNK_ROWS

    def start_gather(b, t):
        pltpu.async_copy(
            inp_hbm.at[pl.ds(r0_of(t), CHUNK_ROWS), :], win_v.at[b], gsem.at[b]
        )

    start_gather(0, 0)
    start_gather(1, 1)

    def chunk(k, carry):
        for b in range(2):
            t = 2 * k + b
            pltpu.make_async_copy(
                inp_hbm.at[pl.ds(r0_of(t), CHUNK_ROWS), :],
                win_v.at[b],
                gsem.at[b],
            ).wait()

            @pl.when(t >= 2)
            def _():
                pltpu.make_async_copy(
                    out_v.at[b],
                    out_hbm.at[:, pl.ds(0, CHUNK_ROWS), :],
                    ssem.at[b],
                ).wait()

            @plsc.parallel_loop(0, N_SLICES)
            def one_slice(i):
                s = idx_v[pl.ds(i, 16)][0]
                base = pl.multiple_of(s & ~15, 16)
                rem = s & 15
                ia_lo = (lane + rem) & 15
                from_a = (lane + rem) < 16
                for r in range(CHUNK_ROWS):
                    a = win_v[b, r, pl.ds(base, 16)]
                    ga = _pick(a, ia_lo)
                    for j in range(VECS):
                        bvec = win_v[
                            b, r, pl.ds(pl.multiple_of(base + 16 * j + 16, 16), 16)
                        ]
                        gb = _pick(bvec, ia_lo)
                        out_v[b, i, r, pl.ds(16 * j, 16)] = jnp.where(
                            from_a, ga, gb
                        )
                        ga = gb

            pltpu.async_copy(
                out_v.at[b],
                out_hbm.at[:, pl.ds(r0_of(t), CHUNK_ROWS), :],
                ssem.at[b],
            )

            @pl.when(t + 2 < CHUNKS_PER_WORKER)
            def _():
                start_gather(b, t + 2)

        return carry

    lax.fori_loop(0, CHUNKS_PER_WORKER // 2, chunk, 0)

    for b in range(2):
        pltpu.make_async_copy(
            out_v.at[b],
            out_hbm.at[:, pl.ds(0, CHUNK_ROWS), :],
            ssem.at[b],
        ).wait()


def kernel(input_tensor, slices_index, slice_len):
    idx_padded = jnp.zeros((IDX_PAD,), jnp.int32).at[:N_SLICES].set(slices_index)
    mesh = plsc.VectorSubcoreMesh(core_axis_name="c", subcore_axis_name="s")
    run = pl.kernel(
        _slice_body,
        out_type=jax.ShapeDtypeStruct((N_SLICES, N_ROWS, SLICE_LEN), jnp.float32),
        mesh=mesh,
        scratch_types=[
            pltpu.VMEM((IDX_PAD,), jnp.int32),
            pltpu.VMEM((2, CHUNK_ROWS, N_COLS), jnp.float32),
            pltpu.VMEM((2, N_SLICES, CHUNK_ROWS, SLICE_LEN), jnp.float32),
            pltpu.SemaphoreType.DMA((2,)),
            pltpu.SemaphoreType.DMA((2,)),
        ],
    )
    return run(input_tensor, idx_padded)


# speedup vs baseline: 1.1128x; 1.0058x over previous
import functools

import jax
import jax.numpy as jnp
from jax import lax
from jax.experimental import pallas as pl
from jax.experimental.pallas import tpu as pltpu, tpu_sc as plsc

N_ROWS = 16384
N_COLS = 3328
N_SLICES = 26
SLICE_LEN = 128
VECS = SLICE_LEN // 16
CHUNK_ROWS = 8
NUM_WORKERS = 32
CHUNKS_PER_WORKER = N_ROWS // (NUM_WORKERS * CHUNK_ROWS)
IDX_PAD = 48


def _pick(vec, idx):
    return lax.gather(
        vec,
        idx[:, None],
        lax.GatherDimensionNumbers(
            offset_dims=(), collapsed_slice_dims=(0,), start_index_map=(0,)
        ),
        (1,),
        mode=lax.GatherScatterMode.PROMISE_IN_BOUNDS,
    )


def _slice_body(inp_hbm, idx_hbm, out_hbm, idx_v, win_v, out_v, gsem, ssem):
    wid = lax.axis_index("s") * 2 + lax.axis_index("c")
    pltpu.sync_copy(idx_hbm, idx_v.at[pl.ds(0, N_SLICES)])
    row_base = wid * CHUNKS_PER_WORKER * CHUNK_ROWS
    lane = lax.iota(jnp.int32, 16)

    def r0_of(t):
        return row_base + t * CHUNK_ROWS

    def start_gather(b, t):
        pltpu.async_copy(
            inp_hbm.at[pl.ds(r0_of(t), CHUNK_ROWS), :], win_v.at[b], gsem.at[b]
        )

    start_gather(0, 0)
    start_gather(1, 1)

    def chunk(k, carry):
        for b in range(2):
            t = 2 * k + b
            pltpu.make_async_copy(
                inp_hbm.at[pl.ds(r0_of(t), CHUNK_ROWS), :],
                win_v.at[b],
                gsem.at[b],
            ).wait()

            @pl.when(t >= 2)
            def _():
                pltpu.make_async_copy(
                    out_v.at[b],
                    out_hbm.at[:, pl.ds(0, CHUNK_ROWS), :],
                    ssem.at[b],
                ).wait()

            @plsc.parallel_loop(0, N_SLICES)
            def one_slice(i):
                s = idx_v[pl.ds(i, 16)][0]
                base = pl.multiple_of(s & ~15, 16)
                rem = s & 15
                ia_lo = (lane + rem) & 15
                from_a = (lane + rem) < 16
                for r in range(CHUNK_ROWS):
                    a = win_v[b, r, pl.ds(base, 16)]
                    ga = _pick(a, ia_lo)
                    for j in range(VECS):
                        bvec = win_v[
                            b, r, pl.ds(pl.multiple_of(base + 16 * j + 16, 16), 16)
                        ]
                        gb = _pick(bvec, ia_lo)
                        out_v[b, i, r, pl.ds(16 * j, 16)] = jnp.where(
                            from_a, ga, gb
                        )
                        ga = gb

            pltpu.async_copy(
                out_v.at[b],
                out_hbm.at[:, pl.ds(r0_of(t), CHUNK_ROWS), :],
                ssem.at[b],
            )

            @pl.when(t + 2 < CHUNKS_PER_WORKER)
            def _():
                start_gather(b, t + 2)

        return carry

    lax.fori_loop(0, CHUNKS_PER_WORKER // 2, chunk, 0)

    for b in range(2):
        pltpu.make_async_copy(
            out_v.at[b],
            out_hbm.at[:, pl.ds(0, CHUNK_ROWS), :],
            ssem.at[b],
        ).wait()


def kernel(input_tensor, slices_index, slice_len):
    mesh = plsc.VectorSubcoreMesh(core_axis_name="c", subcore_axis_name="s")
    run = pl.kernel(
        _slice_body,
        out_type=jax.ShapeDtypeStruct((N_SLICES, N_ROWS, SLICE_LEN), jnp.float32),
        mesh=mesh,
        scratch_types=[
            pltpu.VMEM((IDX_PAD,), jnp.int32),
            pltpu.VMEM((2, CHUNK_ROWS, N_COLS), jnp.float32),
            pltpu.VMEM((2, N_SLICES, CHUNK_ROWS, SLICE_LEN), jnp.float32),
            pltpu.SemaphoreType.DMA((2,)),
            pltpu.SemaphoreType.DMA((2,)),
        ],
    )
    return run(input_tensor, slices_index)
